# Initial kernel scaffold; baseline (speedup 1.0000x reference)
import jax
import jax.numpy as jnp
from jax.experimental import pallas as pl


def kernel(x, pw1, pb1, g1, be1, pw2, pb2, g2, be2, dw, sw, gs, bs):
    raise NotImplementedError("write your pallas kernel here")


# speedup vs baseline: 3.3323x; 3.3323x over previous
import functools
import jax
import jax.numpy as jnp
from jax.experimental import pallas as pl
from jax.experimental.pallas import tpu as pltpu

NEG_SLOPE = 0.01
EPS = 1e-5
N_BINS = 17
BR = 64


def _lrelu(x):
    return jnp.where(x >= 0, x, NEG_SLOPE * x)


def _win11_ax0(a):
    n = a.shape[0]
    t2 = a[: n - 1] + a[1:]
    t4 = t2[: n - 3] + t2[2 : n - 1]
    t8 = t4[: n - 7] + t4[4 : n - 3]
    return t8[: n - 10] + t2[8 : n - 2] + a[10:]


def _win11_ax1(a):
    n = a.shape[1]
    t2 = a[:, : n - 1] + a[:, 1:]
    t4 = t2[:, : n - 3] + t2[:, 2 : n - 1]
    t8 = t4[:, : n - 7] + t4[:, 4 : n - 3]
    return t8[:, : n - 10] + t2[:, 8 : n - 2] + a[:, 10:]


def _reflect_pad5(x):
    h, w = x.shape
    top = jnp.concatenate([x[5 - j : 6 - j] for j in range(5)], axis=0)
    bot = jnp.concatenate([x[h - 2 - j : h - 1 - j] for j in range(5)], axis=0)
    x = jnp.concatenate([top, x, bot], axis=0)
    left = jnp.concatenate([x[:, 5 - j : 6 - j] for j in range(5)], axis=1)
    right = jnp.concatenate([x[:, w - 2 - j : w - 1 - j] for j in range(5)], axis=1)
    return jnp.concatenate([left, x, right], axis=1)


def _k1_body(x_ref, xq_ref):
    H, W = x_ref.shape[2], x_ref.shape[3]
    k = jnp.round(x_ref[0, 0] * 255.0 / 16.0)
    kp = _reflect_pad5(k)
    S = min(64, H)
    for r0 in range(0, H, S):
        seg = kp[r0 : r0 + S + 10]
        best = jnp.zeros((S, W), jnp.float32)
        bestc = jnp.full((S, W), -1.0, jnp.float32)
        for b in range(N_BINS):
            m = (seg == float(b)).astype(jnp.float32)
            cnt = _win11_ax1(_win11_ax0(m))
            take = cnt > bestc
            bestc = jnp.where(take, cnt, bestc)
            best = jnp.where(take, float(b), best)
        xq_ref[0, 0, r0 : r0 + S] = best * (16.0 / 256.0)


def _k1b_body(xq_ref, st_ref):
    C = xq_ref.shape[1]
    sums = [jnp.sum(xq_ref[0, c]) for c in range(C)]
    prods = []
    for c in range(C):
        for c2 in range(C):
            prods.append(jnp.sum(xq_ref[0, c] * xq_ref[0, c2]))
    st_ref[0] = jnp.concatenate(
        [jnp.full((1, 128), s, jnp.float32) for s in sums + prods], axis=0)


def _k2_body(xq_ref, w1_ref, s1_ref, t1_ref, st_ref):
    C = xq_ref.shape[1]
    half = w1_ref.shape[0]
    a1 = []
    for o in range(half):
        h = jnp.zeros(xq_ref.shape[2:], jnp.float32)
        for c in range(C):
            h = h + w1_ref[o, c] * xq_ref[0, c]
        a1.append(_lrelu(h * s1_ref[0, o] + t1_ref[0, o]))
    sums = [jnp.sum(a1[o]) for o in range(half)]
    prods = []
    for o in range(half):
        for o2 in range(o, half):
            prods.append(jnp.sum(a1[o] * a1[o2]))
    st_ref[0, 0] = jnp.concatenate(
        [jnp.full((1, 128), s, jnp.float32) for s in sums + prods], axis=0)


def _k3_body(xq_ref, w1_ref, s1_ref, t1_ref, w2_ref, s2_ref, t2_ref, prep_ref):
    C = xq_ref.shape[1]
    half = w1_ref.shape[0]
    a1 = []
    for o in range(half):
        h = jnp.zeros(xq_ref.shape[2:], jnp.float32)
        for c in range(C):
            h = h + w1_ref[o, c] * xq_ref[0, c]
        a1.append(_lrelu(h * s1_ref[0, o] + t1_ref[0, o]))
    for o in range(half):
        h = jnp.zeros(xq_ref.shape[2:], jnp.float32)
        for c in range(half):
            h = h + w2_ref[o, c] * a1[c]
        prep_ref[0, o] = _lrelu(h * s2_ref[0, o] + t2_ref[0, o])


def _k3b_body(prep_ref, d1_ref, sr_ref, sc_ref, st_ref):
    half = prep_ref.shape[1]
    sp, spp, spz = [], [], []
    sr = sr_ref[...]
    sct = sc_ref[...]
    for g in range(half):
        p = prep_ref[0, g]
        sp.append(jnp.sum(p))
        spp.append(jnp.sum(p * p))
        a = jax.lax.dot_general(p, sct, (((1,), (1,)), ((), ())))
        zr = jnp.dot(sr, d1_ref[0, g])
        spz.append(jnp.sum(a * zr))
    st_ref[0, 0] = jnp.concatenate(
        [jnp.full((1, 128), s, jnp.float32) for s in sp + spp + spz], axis=0)


def _k4_body(prep_ref, sw_ref, sse_ref, sso_ref, tse_ref, tso_ref,
             *drefs, out_ref, heights):
    half = prep_ref.shape[1]
    n_lv = len(heights)
    d_refs = drefs[:n_lv]
    sr_refs = drefs[n_lv : 2 * n_lv]
    sc_refs = drefs[2 * n_lv :]
    blk = prep_ref.shape[2:]
    prev = [prep_ref[0, g] for g in range(half)]
    acc_e = [jnp.zeros(blk, jnp.float32) for _ in range(half)]
    acc_o = [jnp.zeros(blk, jnp.float32) for _ in range(half)]
    for i, h in enumerate(heights):
        for g in range(half):
            if h == 1:
                z = jnp.full(blk, d_refs[i][0, g, 0, 0])
            else:
                t = jnp.dot(sr_refs[i][...], d_refs[i][0, g])
                z = jnp.dot(t, sc_refs[i][...])
            p = prev[g]
            sc_e = sw_ref[g, 0, 0] * p + sw_ref[g, 0, 1] * z
            sc_o = sw_ref[g, 1, 0] * p + sw_ref[g, 1, 1] * z
            acc_e[g] = acc_e[g] + _lrelu(sc_e * sse_ref[i, g] + tse_ref[i, g])
            acc_o[g] = acc_o[g] + _lrelu(sc_o * sso_ref[i, g] + tso_ref[i, g])
            prev[g] = z
    for g in range(half):
        out_ref[0, 2 * g] = acc_e[g]
        out_ref[0, 2 * g + 1] = acc_o[g]


def _sel(H, h):
    ri = (jnp.arange(H) * h) // H
    return (ri[:, None] == jnp.arange(h)[None, :]).astype(jnp.float32)


def kernel(x, pw1, pb1, g1, be1, pw2, pb2, g2, be2, dw, sw, gs, bs):
    B, C, H, W = x.shape
    half = pw1.shape[0]
    N = B * H * W
    par = dict(compiler_params=pltpu.CompilerParams(
        dimension_semantics=("parallel",)))
    par2 = dict(compiler_params=pltpu.CompilerParams(
        dimension_semantics=("parallel", "arbitrary")))

    nst = C + C * C
    xq = pl.pallas_call(
        _k1_body,
        grid=(B, C),
        in_specs=[pl.BlockSpec((1, 1, H, W), lambda b, c: (b, c, 0, 0))],
        out_specs=pl.BlockSpec((1, 1, H, W), lambda b, c: (b, c, 0, 0)),
        out_shape=jax.ShapeDtypeStruct((B, C, H, W), jnp.float32),
        **par2,
    )(x)
    st1 = pl.pallas_call(
        _k1b_body,
        grid=(B,),
        in_specs=[pl.BlockSpec((1, C, H, W), lambda b: (b, 0, 0, 0))],
        out_specs=pl.BlockSpec((1, nst, 128), lambda b: (b, 0, 0)),
        out_shape=jax.ShapeDtypeStruct((B, nst, 128), jnp.float32),
        **par,
    )(xq)

    st1 = jnp.sum(st1[:, :, 0], axis=0)
    mx = st1[:C] / N
    exx = st1[C:].reshape(C, C) / N
    m1 = pw1 @ mx + pb1
    e2 = jnp.einsum('oc,cd,od->o', pw1, exx, pw1) + 2.0 * (pw1 @ mx) * pb1 + pb1 * pb1
    v1 = e2 - m1 * m1
    s1 = g1 * jax.lax.rsqrt(v1 + EPS)
    t1 = be1 - m1 * s1

    npair = half * (half + 1) // 2
    BR2 = min(128, H)
    nrb2 = H // BR2
    st2 = pl.pallas_call(
        _k2_body,
        grid=(B, nrb2),
        in_specs=[pl.BlockSpec((1, C, BR2, W), lambda b, r: (b, 0, r, 0)),
                  pl.BlockSpec((half, C), lambda b, r: (0, 0)),
                  pl.BlockSpec((1, half), lambda b, r: (0, 0)),
                  pl.BlockSpec((1, half), lambda b, r: (0, 0))],
        out_specs=pl.BlockSpec((1, 1, half + npair, 128), lambda b, r: (b, r, 0, 0)),
        out_shape=jax.ShapeDtypeStruct((B, nrb2, half + npair, 128), jnp.float32),
        **par2,
    )(xq, pw1, s1.reshape(1, half), t1.reshape(1, half))

    st2 = jnp.sum(st2[:, :, :, 0], axis=(0, 1))
    ma = st2[:half] / N
    iu, ju = jnp.triu_indices(half)
    eaa = jnp.zeros((half, half)).at[iu, ju].set(st2[half:] / N)
    eaa = eaa + eaa.T - jnp.diag(jnp.diag(eaa))
    m2 = pw2 @ ma + pb2
    e2b = jnp.einsum('oc,cd,od->o', pw2, eaa, pw2) + 2.0 * (pw2 @ ma) * pb2 + pb2 * pb2
    v2 = e2b - m2 * m2
    s2 = g2 * jax.lax.rsqrt(v2 + EPS)
    t2 = be2 - m2 * s2

    prep = pl.pallas_call(
        _k3_body,
        grid=(B, nrb2),
        in_specs=[pl.BlockSpec((1, C, BR2, W), lambda b, r: (b, 0, r, 0)),
                  pl.BlockSpec((half, C), lambda b, r: (0, 0)),
                  pl.BlockSpec((1, half), lambda b, r: (0, 0)),
                  pl.BlockSpec((1, half), lambda b, r: (0, 0)),
                  pl.BlockSpec((half, half), lambda b, r: (0, 0)),
                  pl.BlockSpec((1, half), lambda b, r: (0, 0)),
                  pl.BlockSpec((1, half), lambda b, r: (0, 0))],
        out_specs=pl.BlockSpec((1, half, BR2, W), lambda b, r: (b, 0, r, 0)),
        out_shape=jax.ShapeDtypeStruct((B, half, H, W), jnp.float32),
        **par2,
    )(xq, pw1, s1.reshape(1, half), t1.reshape(1, half),
      pw2, s2.reshape(1, half), t2.reshape(1, half))

    dn = ('NCHW', 'OIHW', 'NCHW')
    ds = []
    d = prep
    while min(d.shape[2], d.shape[3]) >= 3:
        d = jax.lax.conv_general_dilated(d, dw, (3, 3), ((1, 1), (1, 1)),
                                         dimension_numbers=dn)
        ds.append(d)
    heights = tuple(dd.shape[2] for dd in ds)
    n_lv = len(ds)
    srs = [_sel(H, h) for h in heights]
    scs = [_sel(W, h).T for h in heights]

    h1w = heights[0]
    nrb = H // BR
    st3 = pl.pallas_call(
        _k3b_body,
        grid=(B, nrb),
        in_specs=[pl.BlockSpec((1, half, BR, W), lambda b, r: (b, 0, r, 0)),
                  pl.BlockSpec((1, half, h1w, h1w), lambda b, r: (b, 0, 0, 0)),
                  pl.BlockSpec((BR, h1w), lambda b, r: (r, 0)),
                  pl.BlockSpec((h1w, W), lambda b, r: (0, 0))],
        out_specs=pl.BlockSpec((1, 1, 3 * half, 128), lambda b, r: (b, r, 0, 0)),
        out_shape=jax.ShapeDtypeStruct((B, nrb, 3 * half, 128), jnp.float32),
        **par2,
    )(prep, ds[0], srs[0], scs[0])

    st3 = jnp.sum(st3[:, :, :, 0], axis=(0, 1))
    sp1, spp1, spz1 = st3[:half], st3[half:2 * half], st3[2 * half:]

    cr = [s.sum(axis=0) for s in srs]
    cc = [s.sum(axis=1) for s in scs]
    Sp, Sz, Spp, Szz, Spz = [], [], [], [], []
    for i in range(n_lv):
        z = ds[i]
        Sz.append(jnp.einsum('a,kcab,b->c', cr[i], z, cc[i]))
        Szz.append(jnp.einsum('a,kcab,b->c', cr[i], z * z, cc[i]))
        if i == 0:
            Sp.append(sp1)
            Spp.append(spp1)
            Spz.append(spz1)
        else:
            p = ds[i - 1]
            Sp.append(jnp.einsum('a,kcab,b->c', cr[i - 1], p, cc[i - 1]))
            Spp.append(jnp.einsum('a,kcab,b->c', cr[i - 1], p * p, cc[i - 1]))
            jr = srs[i - 1].T @ srs[i]
            jc = scs[i - 1] @ scs[i].T
            t = jnp.einsum('kcab,aA->kcAb', p, jr)
            t = jnp.einsum('kcAb,bB->kcAB', t, jc)
            Spz.append(jnp.einsum('kcAB,kcAB->c', t, ds[i]))
    sse, sso, tse, tso = [], [], [], []
    for i in range(n_lv):
        for o, (ss_l, ts_l) in ((0, (sse, tse)), (1, (sso, tso))):
            w0, w1 = sw[:, o, 0], sw[:, o, 1]
            m = (w0 * Sp[i] + w1 * Sz[i]) / N
            e2c = (w0 * w0 * Spp[i] + 2 * w0 * w1 * Spz[i] + w1 * w1 * Szz[i]) / N
            v = e2c - m * m
            sc_ = gs[o::2] * jax.lax.rsqrt(v + EPS)
            ss_l.append(sc_)
            ts_l.append(bs[o::2] - m * sc_)
    sse = jnp.stack(sse); sso = jnp.stack(sso)
    tse = jnp.stack(tse); tso = jnp.stack(tso)

    body = functools.partial(_k4_body, heights=heights)

    def k4(*refs):
        body(*refs[:-1], out_ref=refs[-1])

    in_specs = [pl.BlockSpec((1, half, BR, W), lambda b, r: (b, 0, r, 0)),
                pl.BlockSpec((half, 2, 2), lambda b, r: (0, 0, 0)),
                pl.BlockSpec((n_lv, half), lambda b, r: (0, 0)),
                pl.BlockSpec((n_lv, half), lambda b, r: (0, 0)),
                pl.BlockSpec((n_lv, half), lambda b, r: (0, 0)),
                pl.BlockSpec((n_lv, half), lambda b, r: (0, 0))]
    args = [prep, sw, sse, sso, tse, tso]
    for i in range(n_lv):
        h = heights[i]
        in_specs.append(pl.BlockSpec((1, half, h, h), lambda b, r: (b, 0, 0, 0)))
        args.append(ds[i])
    for i in range(n_lv):
        h = heights[i]
        in_specs.append(pl.BlockSpec((BR, h), lambda b, r: (r, 0)))
        args.append(srs[i])
    for i in range(n_lv):
        h = heights[i]
        in_specs.append(pl.BlockSpec((h, W), lambda b, r: (0, 0)))
        args.append(scs[i])

    score = pl.pallas_call(
        k4,
        grid=(B, nrb),
        in_specs=in_specs,
        out_specs=pl.BlockSpec((1, 2 * half, BR, W), lambda b, r: (b, 0, r, 0)),
        out_shape=jax.ShapeDtypeStruct((B, 2 * half, H, W), jnp.float32),
        **par2,
    )(*args)
    return score
